# baseline (speedup 1.0000x reference)
import jax
import jax.numpy as jnp
from jax.experimental import pallas as pl
from jax.experimental.pallas import tpu as pltpu

H = 12


def kernel(target_emb, neighbor_embs, swing_scores, Wq, Wk, Wv, Wo,
           swing_scale, ln_gamma, ln_beta):
    B, D = target_emb.shape
    N = neighbor_embs.shape[1]
    hd = D // H
    Bb = 128
    SB = 8
    nsub = Bb // SB
    f32 = jnp.float32

    def _fused(t_ref, nbt_ref, sw_ref, wq_ref, wk_ref, wv_ref,
               wo_ref, scale_ref, gamma_ref, beta_ref, o_ref):
        t = t_ref[...]
        q = jax.lax.dot_general(t, wq_ref[...], (((1,), (1,)), ((), ())),
                                preferred_element_type=f32)
        a_heads = [jnp.dot(q[:, h * hd:(h + 1) * hd],
                           wk_ref[h * hd:(h + 1) * hd, :],
                           preferred_element_type=f32) for h in range(H)]
        scale = scale_ref[0, 0]
        col = jax.lax.broadcasted_iota(jnp.int32, (1, SB, H * SB), 2)
        row = jax.lax.broadcasted_iota(jnp.int32, (1, SB, H * SB), 1)
        own = (col % SB) == row

        nb_slabs, scores_l = [], []
        for s in range(nsub):
            nb_ts = nbt_ref[:, s * SB:(s + 1) * SB, :]
            nb_slabs.append(nb_ts)
            a_sub = jnp.concatenate(
                [a_heads[h][s * SB:(s + 1) * SB] for h in range(H)],
                axis=0)
            scores = jax.lax.dot_general(
                nb_ts, a_sub, (((2,), (1,)), ((), ())),
                preferred_element_type=f32) * (hd ** -0.5)
            sw_ts = sw_ref[s * SB:(s + 1) * SB, :].T
            scores_l.append(scores + (scale * sw_ts)[:, :, None])

        wms = []
        for s in range(nsub):
            scores = scores_l[s]
            mx = jnp.max(scores, axis=0, keepdims=True)
            e = jnp.exp(scores - mx)
            w = e / jnp.sum(e, axis=0, keepdims=True)
            wm = jnp.where(own, w, 0.0)
            wms.append(wm.reshape(N * SB, H * SB))

        m_pieces = [
            jax.lax.dot_general(
                wms[s], nb_slabs[s].reshape(N * SB, D),
                (((0,), (0,)), ((), ())),
                preferred_element_type=f32)
            for s in range(nsub)]

        ao_parts = []
        for h in range(H):
            m_h = jnp.concatenate(
                [m_pieces[s][h * SB:(h + 1) * SB] for s in range(nsub)],
                axis=0)
            ao_parts.append(jax.lax.dot_general(
                m_h, wv_ref[h * hd:(h + 1) * hd, :],
                (((1,), (1,)), ((), ())),
                preferred_element_type=f32))
        ao = jnp.concatenate(ao_parts, axis=1)

        y = t + jax.lax.dot_general(ao, wo_ref[...],
                                    (((1,), (1,)), ((), ())),
                                    preferred_element_type=f32)
        mu = jnp.mean(y, axis=-1, keepdims=True)
        yc = y - mu
        var = jnp.mean(yc * yc, axis=-1, keepdims=True)
        o_ref[...] = (yc * jax.lax.rsqrt(var + 1e-5) * gamma_ref[...]
                      + beta_ref[...])

    nbt = neighbor_embs.transpose(1, 0, 2)
    scale2 = swing_scale.reshape(1, 1)
    gamma2 = ln_gamma.reshape(1, D)
    beta2 = ln_beta.reshape(1, D)

    return pl.pallas_call(
        _fused,
        grid=(B // Bb,),
        in_specs=[
            pl.BlockSpec((Bb, D), lambda i: (i, 0)),
            pl.BlockSpec((N, Bb, D), lambda i: (0, i, 0)),
            pl.BlockSpec((Bb, N), lambda i: (i, 0)),
            pl.BlockSpec((D, D), lambda i: (0, 0)),
            pl.BlockSpec((D, D), lambda i: (0, 0)),
            pl.BlockSpec((D, D), lambda i: (0, 0)),
            pl.BlockSpec((D, D), lambda i: (0, 0)),
            pl.BlockSpec((1, 1), lambda i: (0, 0)),
            pl.BlockSpec((1, D), lambda i: (0, 0)),
            pl.BlockSpec((1, D), lambda i: (0, 0)),
        ],
        out_specs=pl.BlockSpec((Bb, D), lambda i: (i, 0)),
        out_shape=jax.ShapeDtypeStruct((B, D), jnp.float32),
        compiler_params=pltpu.CompilerParams(
            dimension_semantics=("parallel",),
            vmem_limit_bytes=120 * 1024 * 1024),
    )(target_emb, nbt, swing_scores, Wq, Wk, Wv, Wo,
      scale2, gamma2, beta2)


# speedup vs baseline: 1.0142x; 1.0142x over previous
import jax
import jax.numpy as jnp
from jax.experimental import pallas as pl
from jax.experimental.pallas import tpu as pltpu

H = 12


def kernel(target_emb, neighbor_embs, swing_scores, Wq, Wk, Wv, Wo,
           swing_scale, ln_gamma, ln_beta):
    B, D = target_emb.shape
    N = neighbor_embs.shape[1]
    hd = D // H
    Bb = 128
    SB = 8
    nsub = Bb // SB
    f32 = jnp.float32

    def _fused(t_ref, nbt_ref, sw_ref, wq_ref, wk_ref, wv_ref,
               wo_ref, scale_ref, gamma_ref, beta_ref, o_ref):
        t = t_ref[...]
        q = jax.lax.dot_general(t, wq_ref[...], (((1,), (1,)), ((), ())),
                                preferred_element_type=f32)
        a_heads = [jnp.dot(q[:, h * hd:(h + 1) * hd],
                           wk_ref[h * hd:(h + 1) * hd, :],
                           preferred_element_type=f32) for h in range(H)]
        scale = scale_ref[0, 0]
        col = jax.lax.broadcasted_iota(jnp.int32, (1, SB, H * SB), 2)
        row = jax.lax.broadcasted_iota(jnp.int32, (1, SB, H * SB), 1)
        own = (col % SB) == row

        nb_slabs, scores_l = [], []
        for s in range(nsub):
            nb_ts = nbt_ref[:, s * SB:(s + 1) * SB, :]
            nb16 = nb_ts.astype(jnp.bfloat16)
            nb_slabs.append(nb16)
            a_sub = jnp.concatenate(
                [a_heads[h][s * SB:(s + 1) * SB] for h in range(H)],
                axis=0)
            scores = jax.lax.dot_general(
                nb16, a_sub.astype(jnp.bfloat16), (((2,), (1,)), ((), ())),
                preferred_element_type=f32) * (hd ** -0.5)
            sw_ts = sw_ref[s * SB:(s + 1) * SB, :].T
            scores_l.append(scores + (scale * sw_ts)[:, :, None])

        wms = []
        for s in range(nsub):
            scores = scores_l[s]
            mx = jnp.max(scores, axis=0, keepdims=True)
            e = jnp.exp(scores - mx)
            w = e / jnp.sum(e, axis=0, keepdims=True)
            wm = jnp.where(own, w, 0.0)
            wms.append(wm.reshape(N * SB, H * SB).astype(jnp.bfloat16))

        m_pieces = [
            jax.lax.dot_general(
                wms[s], nb_slabs[s].reshape(N * SB, D),
                (((0,), (0,)), ((), ())),
                preferred_element_type=f32)
            for s in range(nsub)]

        ao_parts = []
        for h in range(H):
            m_h = jnp.concatenate(
                [m_pieces[s][h * SB:(h + 1) * SB] for s in range(nsub)],
                axis=0)
            ao_parts.append(jax.lax.dot_general(
                m_h, wv_ref[h * hd:(h + 1) * hd, :],
                (((1,), (1,)), ((), ())),
                preferred_element_type=f32))
        ao = jnp.concatenate(ao_parts, axis=1)

        y = t + jax.lax.dot_general(ao, wo_ref[...],
                                    (((1,), (1,)), ((), ())),
                                    preferred_element_type=f32)
        mu = jnp.mean(y, axis=-1, keepdims=True)
        yc = y - mu
        var = jnp.mean(yc * yc, axis=-1, keepdims=True)
        o_ref[...] = (yc * jax.lax.rsqrt(var + 1e-5) * gamma_ref[...]
                      + beta_ref[...])

    nbt = neighbor_embs.transpose(1, 0, 2)
    scale2 = swing_scale.reshape(1, 1)
    gamma2 = ln_gamma.reshape(1, D)
    beta2 = ln_beta.reshape(1, D)

    return pl.pallas_call(
        _fused,
        grid=(B // Bb,),
        in_specs=[
            pl.BlockSpec((Bb, D), lambda i: (i, 0)),
            pl.BlockSpec((N, Bb, D), lambda i: (0, i, 0)),
            pl.BlockSpec((Bb, N), lambda i: (i, 0)),
            pl.BlockSpec((D, D), lambda i: (0, 0)),
            pl.BlockSpec((D, D), lambda i: (0, 0)),
            pl.BlockSpec((D, D), lambda i: (0, 0)),
            pl.BlockSpec((D, D), lambda i: (0, 0)),
            pl.BlockSpec((1, 1), lambda i: (0, 0)),
            pl.BlockSpec((1, D), lambda i: (0, 0)),
            pl.BlockSpec((1, D), lambda i: (0, 0)),
        ],
        out_specs=pl.BlockSpec((Bb, D), lambda i: (i, 0)),
        out_shape=jax.ShapeDtypeStruct((B, D), jnp.float32),
        compiler_params=pltpu.CompilerParams(
            dimension_semantics=("parallel",),
            vmem_limit_bytes=120 * 1024 * 1024),
    )(target_emb, nbt, swing_scores, Wq, Wk, Wv, Wo,
      scale2, gamma2, beta2)
